# Initial kernel scaffold; baseline (speedup 1.0000x reference)
import jax
import jax.numpy as jnp
from jax.experimental import pallas as pl


def kernel(seq_items, emb_table, W_ih, W_hh, b_ih, b_hh):
    raise NotImplementedError("write your pallas kernel here")


# speedup vs baseline: 1.0234x; 1.0234x over previous
import functools

import jax
import jax.numpy as jnp
from jax import lax
from jax.experimental import pallas as pl
from jax.experimental.pallas import tpu as pltpu
from jax.experimental.pallas import tpu_sc as plsc


_CHUNK = 128


def _sc_gather_body(n_chunks, idx_hbm, table_hbm, out_hbm, idx_v, rows_v, sem):
    nc = plsc.get_sparse_core_info().num_subcores
    w = lax.axis_index("c") * nc + lax.axis_index("s")
    per_w = n_chunks * _CHUNK
    pltpu.sync_copy(idx_hbm.at[w], idx_v)

    def body(j, carry):
        pltpu.async_copy(table_hbm.at[idx_v.at[j]], rows_v, sem).wait()
        pltpu.sync_copy(rows_v, out_hbm.at[pl.ds(w * per_w + j * _CHUNK, _CHUNK)])
        return carry

    lax.fori_loop(0, n_chunks, body, 0)


def _sc_gather(idx_flat, table):
    n = idx_flat.shape[0]
    e = table.shape[1]
    info = plsc.get_sparse_core_info()
    nw = info.num_cores * info.num_subcores
    n_chunks = n // (nw * _CHUNK)
    idx_r = idx_flat.reshape(nw, n_chunks, _CHUNK)
    mesh = plsc.VectorSubcoreMesh(core_axis_name="c", subcore_axis_name="s")
    fn = pl.kernel(
        functools.partial(_sc_gather_body, n_chunks),
        out_type=jax.ShapeDtypeStruct((n, e), jnp.float32),
        mesh=mesh,
        scratch_types=[
            pltpu.VMEM((n_chunks, _CHUNK), jnp.int32),
            pltpu.VMEM((_CHUNK, e), jnp.float32),
            pltpu.SemaphoreType.DMA,
        ],
        compiler_params=pltpu.CompilerParams(use_tc_tiling_on_sc=False),
    )
    return fn(idx_r, table)


def _gru_body(n_steps, emb_ref, wih_ref, whh_ref, bih_ref, bhh_ref, out_ref,
              h_ref):
    t = pl.program_id(0)

    @pl.when(t == 0)
    def _init():
        h_ref[...] = jnp.zeros_like(h_ref)

    x = emb_ref[0]
    h = h_ref[...]
    e = h.shape[0]
    gi = lax.dot_general(wih_ref[...], x, (((1,), (1,)), ((), ())),
                         preferred_element_type=jnp.float32) + bih_ref[...]
    gh = lax.dot_general(whh_ref[...], h, (((1,), (0,)), ((), ())),
                         preferred_element_type=jnp.float32) + bhh_ref[...]
    r = jax.nn.sigmoid(gi[0:e] + gh[0:e])
    z = jax.nn.sigmoid(gi[e:2 * e] + gh[e:2 * e])
    nn = jnp.tanh(gi[2 * e:3 * e] + r * gh[2 * e:3 * e])
    hn = (1.0 - z) * nn + z * h
    h_ref[...] = hn

    @pl.when(t == n_steps - 1)
    def _out():
        out_ref[...] = hn


def _gru_tc(emb, w_ih, w_hh, b_ih, b_hh):
    seq_len, batch, e = emb.shape
    g = w_ih.shape[0]
    grid = (seq_len,)
    out = pl.pallas_call(
        functools.partial(_gru_body, seq_len),
        grid=grid,
        in_specs=[
            pl.BlockSpec((1, batch, e), lambda t: (t, 0, 0)),
            pl.BlockSpec((g, e), lambda t: (0, 0)),
            pl.BlockSpec((g, e), lambda t: (0, 0)),
            pl.BlockSpec((g, 1), lambda t: (0, 0)),
            pl.BlockSpec((g, 1), lambda t: (0, 0)),
        ],
        out_specs=pl.BlockSpec((e, batch), lambda t: (0, 0)),
        out_shape=jax.ShapeDtypeStruct((e, batch), jnp.float32),
        scratch_shapes=[pltpu.VMEM((e, batch), jnp.float32)],
        compiler_params=pltpu.CompilerParams(
            dimension_semantics=("arbitrary",)),
    )(emb, w_ih, w_hh, b_ih.reshape(g, 1), b_hh.reshape(g, 1))
    return out


def kernel(seq_items, emb_table, W_ih, W_hh, b_ih, b_hh):
    batch, seq_len = seq_items.shape
    e = emb_table.shape[1]
    idx_flat = seq_items.astype(jnp.int32).T.reshape(-1)
    gathered = _sc_gather(idx_flat, emb_table)
    emb = gathered.reshape(seq_len, batch, e)
    h = _gru_tc(emb, W_ih, W_hh, b_ih, b_hh)
    return h.T


# speedup vs baseline: 1.2024x; 1.1749x over previous
import functools

import jax
import jax.numpy as jnp
from jax import lax
from jax.experimental import pallas as pl
from jax.experimental.pallas import tpu as pltpu
from jax.experimental.pallas import tpu_sc as plsc


_CHUNK = 128
_NBUF = 5


def _sc_gather_body(n_chunks, idx_hbm, table_hbm, out_hbm, idx_v, rows_v,
                    gsems):
    nc = plsc.get_sparse_core_info().num_subcores
    w = lax.axis_index("c") * nc + lax.axis_index("s")
    per_w = n_chunks * _CHUNK
    pltpu.sync_copy(idx_hbm.at[w], idx_v)

    def gather_start(j, b):
        pltpu.async_copy(table_hbm.at[idx_v.at[j]], rows_v.at[b], gsems.at[b])

    def gather_wait(j, b):
        pltpu.make_async_copy(table_hbm.at[idx_v.at[j]], rows_v.at[b],
                              gsems.at[b]).wait()

    for b in range(_NBUF):
        gather_start(b, b)

    def body(i0, carry):
        for b in range(_NBUF):
            j = i0 + b
            gather_wait(j, b)
            pltpu.sync_copy(rows_v.at[b],
                            out_hbm.at[pl.ds(w * per_w + j * _CHUNK, _CHUNK)])
            nxt = j + _NBUF

            @pl.when(nxt < n_chunks)
            def _():
                gather_start(nxt, b)
        return carry

    lax.fori_loop(0, n_chunks // _NBUF, lambda i, c: body(i * _NBUF, c), 0,
                  unroll=False)


def _sc_gather(idx_flat, table):
    n = idx_flat.shape[0]
    e = table.shape[1]
    info = plsc.get_sparse_core_info()
    nw = info.num_cores * info.num_subcores
    n_chunks = n // (nw * _CHUNK)
    idx_r = idx_flat.reshape(nw, n_chunks, _CHUNK)
    mesh = plsc.VectorSubcoreMesh(core_axis_name="c", subcore_axis_name="s")
    fn = pl.kernel(
        functools.partial(_sc_gather_body, n_chunks),
        out_type=jax.ShapeDtypeStruct((n, e), jnp.float32),
        mesh=mesh,
        scratch_types=[
            pltpu.VMEM((n_chunks, _CHUNK), jnp.int32),
            pltpu.VMEM((_NBUF, _CHUNK, e), jnp.float32),
            pltpu.SemaphoreType.DMA((_NBUF,)),
        ],
        compiler_params=pltpu.CompilerParams(use_tc_tiling_on_sc=False),
    )
    return fn(idx_r, table)


def _gru_body(n_steps, e2, emb_ref, wih_ref, whh_ref, bih_ref, bhh_ref,
              out_ref, h_ref):
    t = pl.program_id(0)

    @pl.when(t == 0)
    def _init():
        h_ref[...] = jnp.zeros_like(h_ref)

    x = emb_ref[0]
    h = h_ref[...]
    gi = jnp.dot(x, wih_ref[...],
                 preferred_element_type=jnp.float32) + bih_ref[...]
    gh = jnp.dot(h, whh_ref[...],
                 preferred_element_type=jnp.float32) + bhh_ref[...]
    r = jax.nn.sigmoid(gi[:, 0:e2] + gh[:, 0:e2])
    z = jax.nn.sigmoid(gi[:, e2:2 * e2] + gh[:, e2:2 * e2])
    nn = jnp.tanh(gi[:, 2 * e2:3 * e2] + r * gh[:, 2 * e2:3 * e2])
    hn = (1.0 - z) * nn + z * h
    h_ref[...] = hn

    @pl.when(t == n_steps - 1)
    def _out():
        out_ref[...] = hn


def _pair_block_weights(w, e):
    wt = w.T
    eye2 = jnp.eye(2, dtype=w.dtype)
    blocks = [jnp.kron(eye2, wt[:, g * e:(g + 1) * e]) for g in range(3)]
    return jnp.concatenate(blocks, axis=1)


def _pair_bias(b, e):
    return jnp.concatenate(
        [jnp.tile(b[g * e:(g + 1) * e], 2) for g in range(3)]).reshape(1, -1)


def _gru_tc(emb2, w_ih, w_hh, b_ih, b_hh, e):
    seq_len, half_b, e2 = emb2.shape
    g2 = 6 * e
    wb_ih = _pair_block_weights(w_ih, e)
    wb_hh = _pair_block_weights(w_hh, e)
    bb_ih = _pair_bias(b_ih, e)
    bb_hh = _pair_bias(b_hh, e)
    out = pl.pallas_call(
        functools.partial(_gru_body, seq_len, e2),
        grid=(seq_len,),
        in_specs=[
            pl.BlockSpec((1, half_b, e2), lambda t: (t, 0, 0)),
            pl.BlockSpec((e2, g2), lambda t: (0, 0)),
            pl.BlockSpec((e2, g2), lambda t: (0, 0)),
            pl.BlockSpec((1, g2), lambda t: (0, 0)),
            pl.BlockSpec((1, g2), lambda t: (0, 0)),
        ],
        out_specs=pl.BlockSpec((half_b, e2), lambda t: (0, 0)),
        out_shape=jax.ShapeDtypeStruct((half_b, e2), jnp.float32),
        scratch_shapes=[pltpu.VMEM((half_b, e2), jnp.float32)],
        compiler_params=pltpu.CompilerParams(
            dimension_semantics=("arbitrary",)),
    )(emb2, wb_ih, wb_hh, bb_ih, bb_hh)
    return out


def kernel(seq_items, emb_table, W_ih, W_hh, b_ih, b_hh):
    batch, seq_len = seq_items.shape
    e = emb_table.shape[1]
    idx_flat = seq_items.astype(jnp.int32).T.reshape(-1)
    gathered = _sc_gather(idx_flat, emb_table)
    emb2 = gathered.reshape(seq_len, batch // 2, 2 * e)
    h2 = _gru_tc(emb2, W_ih, W_hh, b_ih, b_hh, e)
    return h2.reshape(batch, e)
